# baseline (speedup 1.0000x reference)
import functools

import jax
import jax.numpy as jnp
import numpy as np
from jax import lax
from jax.experimental import pallas as pl
from jax.experimental.pallas import tpu as pltpu
from jax.experimental.pallas import tpu_sc as plsc

_DIM = 128
_BATCH = 128

_IDX1 = np.array([95708, 1475, 98019, 67593, 55310, 15163, 79100, 67173, 62548, 32275, 33196, 34149, 21250, 36283, 61971, 88800, 68851, 33799, 91179, 88747, 53869, 90273, 18768, 77667, 88631, 99005, 85631, 41828, 83884, 5177, 66884, 58995, 94144, 95303, 28243, 8732, 62900, 51257, 61057, 85597, 34510, 55808, 76234, 9154, 69256, 80537, 46330, 29064, 83617, 11568, 34967, 3295, 53459, 78087, 99219, 24592, 71095, 35965, 71656, 42573, 70312, 40499, 26952, 556, 14030, 42684, 97405, 21847, 96077, 16388, 10794, 52120, 8779, 55454, 4748, 79963, 35476, 29708, 81431, 66338, 25302, 43852, 28002, 17837, 44523, 81477, 39332, 52949, 9515, 87583, 61954, 62581, 52269, 13884, 16411, 14945, 34544, 67453, 30385, 89901, 95997, 5357, 16985, 50866, 89191, 35442, 3727, 5715, 42882, 36763, 90722, 18735, 57178, 53372, 69675, 92078, 68990, 27449, 28956, 67056, 117, 3124, 30808, 35078, 165, 88059, 59371, 60879], dtype=np.int32)
_IDX2 = np.array([54893, 78472, 35784, 84508, 44403, 17508, 99241, 70346, 50092, 36631, 45196, 44916, 40104, 95911, 73377, 31764, 99681, 8230, 55825, 99931, 48871, 48318, 75322, 60772, 62226, 57724, 35702, 3446, 39162, 7729, 7290, 56918, 79724, 60035, 82683, 76928, 39882, 70032, 24986, 67950, 70386, 67891, 67630, 94911, 3153, 86948, 97761, 42898, 56260, 72905, 90207, 21540, 94133, 34756, 35256, 11382, 35769, 21540, 9812, 26928, 56109, 81207, 28423, 6329, 45768, 47299, 66045, 8158, 9380, 1414, 53660, 73658, 51804, 33016, 15858, 97999, 88705, 95081, 77432, 73294, 45882, 94487, 52713, 4514, 94693, 57350, 9021, 47119, 27089, 97314, 41505, 44477, 3123, 56297, 57297, 53056, 39950, 62202, 50791, 742, 33604, 49023, 80748, 40812, 30241, 64541, 49479, 56630, 15313, 52176, 43148, 30587, 90654, 30265, 28190, 4074, 23228, 97800, 98209, 29105, 99139, 8834, 26913, 38625, 63327, 39298, 98732, 36249], dtype=np.int32)
_IDX3 = np.array([98961, 7517, 81163, 2096, 57005, 34770, 39411, 38857, 38562, 38435, 58009, 49687, 63736, 83531, 72382, 40002, 49740, 8386, 67037, 40869, 66181, 57428, 52375, 26858, 62789, 36725, 12277, 91284, 93591, 24341, 47093, 82246, 73478, 68284, 83496, 70728, 17981, 69305, 58088, 11884, 8270, 99034, 72614, 53928, 62543, 17367, 74547, 11562, 2482, 45569, 19183, 74797, 12876, 92627, 78491, 57264, 53886, 81039, 45978, 1926, 57116, 94153, 62079, 97962, 20842, 73959, 51012, 46153, 6666, 41219, 383, 41667, 52310, 49870, 57119, 29921, 90459, 51477, 65998, 21101, 48881, 8979, 48160, 83567, 63026, 38282, 92997, 62957, 17886, 26509, 24005, 79325, 79035, 83440, 34717, 50500, 13538, 59858, 81073, 24857, 30673, 89348, 4047, 15922, 91091, 38555, 33409, 5234, 18128, 16223, 59575, 71812, 44240, 33120, 75605, 20468, 3330, 9157, 28535, 48186, 29608, 56843, 96376, 64520, 83903, 30693, 87832, 15506], dtype=np.int32)

_NW = 32
_CROWS = 1024
_CPW = _CROWS // _NW
_RPW = _BATCH // 16

@functools.cache
def _make_sc_gather():
    mesh = plsc.VectorSubcoreMesh(core_axis_name="c", subcore_axis_name="s")

    @functools.partial(
        pl.kernel,
        mesh=mesh,
        compiler_params=pltpu.CompilerParams(use_tc_tiling_on_sc=False),
        out_type=[
            jax.ShapeDtypeStruct((_CROWS, _DIM + 1), jnp.float32),
            jax.ShapeDtypeStruct((_BATCH, _DIM), jnp.float32),
        ],
        scratch_types=[
            pltpu.VMEM((_CPW,), jnp.int32),
            pltpu.VMEM((_CPW, _DIM + 1), jnp.float32),
            pltpu.VMEM((_RPW,), jnp.int32),
            pltpu.VMEM((_RPW, _DIM), jnp.float32),
            pltpu.SemaphoreType.DMA,
        ],
    )
    def _sc_gather(class_hbm, rel_hbm, cidx_hbm, ridx_hbm, out_c, out_r,
                   cidx_v, crows_v, ridx_v, rrows_v, sem):
        wid = lax.axis_index("s") * 2 + lax.axis_index("c")
        base = wid * _CPW
        pltpu.sync_copy(cidx_hbm.at[pl.ds(base, _CPW)], cidx_v)
        pltpu.async_copy(class_hbm.at[cidx_v], crows_v, sem).wait()
        pltpu.sync_copy(crows_v, out_c.at[pl.ds(base, _CPW)])

        @pl.when(wid < 16)
        def _():
            rbase = wid * _RPW
            pltpu.sync_copy(ridx_hbm.at[pl.ds(rbase, _RPW)], ridx_v)
            pltpu.async_copy(rel_hbm.at[ridx_v], rrows_v, sem).wait()
            pltpu.sync_copy(rrows_v, out_r.at[pl.ds(rbase, _RPW)])

    return _sc_gather


def _norm(x):
    return jnp.sqrt(jnp.sum(x * x, axis=1, keepdims=True))


def _math_body(g_ref, rel_ref, out_ref):
    g = g_ref[...]
    cc = g[:, :_DIM]
    r = jnp.abs(g[:, _DIM:_DIM + 1])

    def grp(i):
        return cc[i * _BATCH:(i + 1) * _BATCH], r[i * _BATCH:(i + 1) * _BATCH]

    c1, rc1 = grp(0)
    d1, rd1 = grp(1)
    c2, rc2 = grp(2)
    d2, rd2 = grp(3)
    e2, re2 = grp(4)
    c3, rc3 = grp(5)
    d3, rd3 = grp(6)
    rel = rel_ref[...]

    reg1 = jnp.abs(_norm(c1) - 1.0) + jnp.abs(_norm(d1) - 1.0)
    l1 = jax.nn.relu(_norm(c1 - d1) + rc1 - rd1)
    loss1 = jnp.sum(jnp.sqrt(l1 + reg1)) / _BATCH

    reg2 = (jnp.abs(_norm(c2) - 1.0) + jnp.abs(_norm(d2) - 1.0)
            + jnp.abs(_norm(e2) - 1.0))
    l2 = (jax.nn.relu(_norm(c2 - d2) - rc2 - rd2)
          + jax.nn.relu(_norm(c2 - e2) - rc2)
          + jax.nn.relu(_norm(d2 - e2) - rd2))
    loss2 = jnp.sum(jnp.sqrt(l2 + reg2)) / _BATCH

    reg3 = jnp.abs(_norm(c3) - 1.0) + jnp.abs(_norm(d3) - 1.0)
    l3 = jnp.maximum(0.0, _norm(c3 + rel - d3) + rc3 - rd3)
    loss3 = jnp.sum(jnp.sqrt(l3 + reg3)) / _BATCH

    out_ref[...] = jnp.broadcast_to(loss1 + loss2 + loss3, (1, 1))


def kernel(nf1, nf2, nf3, class_emb, rel_emb):
    d1 = nf1[_IDX1]
    d2 = nf2[_IDX2]
    d3 = nf3[_IDX3]
    cidx = jnp.concatenate([
        d1[:, 0], d1[:, 1],
        d2[:, 0], d2[:, 1], d2[:, 2],
        d3[:, 0], d3[:, 2],
        jnp.zeros((_CROWS - 7 * _BATCH,), jnp.int32),
    ])
    ridx = d3[:, 1]
    gc, gr = _make_sc_gather()(class_emb, rel_emb, cidx, ridx)
    out = pl.pallas_call(
        _math_body,
        out_shape=jax.ShapeDtypeStruct((1, 1), jnp.float32),
    )(gc, gr)
    return jnp.reshape(out, ())


# speedup vs baseline: 6.8534x; 6.8534x over previous
import functools

import jax
import jax.numpy as jnp
import numpy as np
from jax import lax
from jax.experimental import pallas as pl
from jax.experimental.pallas import tpu as pltpu
from jax.experimental.pallas import tpu_sc as plsc

_DIM = 128
_BATCH = 128

_IDX1 = np.array([95708, 1475, 98019, 67593, 55310, 15163, 79100, 67173, 62548, 32275, 33196, 34149, 21250, 36283, 61971, 88800, 68851, 33799, 91179, 88747, 53869, 90273, 18768, 77667, 88631, 99005, 85631, 41828, 83884, 5177, 66884, 58995, 94144, 95303, 28243, 8732, 62900, 51257, 61057, 85597, 34510, 55808, 76234, 9154, 69256, 80537, 46330, 29064, 83617, 11568, 34967, 3295, 53459, 78087, 99219, 24592, 71095, 35965, 71656, 42573, 70312, 40499, 26952, 556, 14030, 42684, 97405, 21847, 96077, 16388, 10794, 52120, 8779, 55454, 4748, 79963, 35476, 29708, 81431, 66338, 25302, 43852, 28002, 17837, 44523, 81477, 39332, 52949, 9515, 87583, 61954, 62581, 52269, 13884, 16411, 14945, 34544, 67453, 30385, 89901, 95997, 5357, 16985, 50866, 89191, 35442, 3727, 5715, 42882, 36763, 90722, 18735, 57178, 53372, 69675, 92078, 68990, 27449, 28956, 67056, 117, 3124, 30808, 35078, 165, 88059, 59371, 60879], dtype=np.int32)
_IDX2 = np.array([54893, 78472, 35784, 84508, 44403, 17508, 99241, 70346, 50092, 36631, 45196, 44916, 40104, 95911, 73377, 31764, 99681, 8230, 55825, 99931, 48871, 48318, 75322, 60772, 62226, 57724, 35702, 3446, 39162, 7729, 7290, 56918, 79724, 60035, 82683, 76928, 39882, 70032, 24986, 67950, 70386, 67891, 67630, 94911, 3153, 86948, 97761, 42898, 56260, 72905, 90207, 21540, 94133, 34756, 35256, 11382, 35769, 21540, 9812, 26928, 56109, 81207, 28423, 6329, 45768, 47299, 66045, 8158, 9380, 1414, 53660, 73658, 51804, 33016, 15858, 97999, 88705, 95081, 77432, 73294, 45882, 94487, 52713, 4514, 94693, 57350, 9021, 47119, 27089, 97314, 41505, 44477, 3123, 56297, 57297, 53056, 39950, 62202, 50791, 742, 33604, 49023, 80748, 40812, 30241, 64541, 49479, 56630, 15313, 52176, 43148, 30587, 90654, 30265, 28190, 4074, 23228, 97800, 98209, 29105, 99139, 8834, 26913, 38625, 63327, 39298, 98732, 36249], dtype=np.int32)
_IDX3 = np.array([98961, 7517, 81163, 2096, 57005, 34770, 39411, 38857, 38562, 38435, 58009, 49687, 63736, 83531, 72382, 40002, 49740, 8386, 67037, 40869, 66181, 57428, 52375, 26858, 62789, 36725, 12277, 91284, 93591, 24341, 47093, 82246, 73478, 68284, 83496, 70728, 17981, 69305, 58088, 11884, 8270, 99034, 72614, 53928, 62543, 17367, 74547, 11562, 2482, 45569, 19183, 74797, 12876, 92627, 78491, 57264, 53886, 81039, 45978, 1926, 57116, 94153, 62079, 97962, 20842, 73959, 51012, 46153, 6666, 41219, 383, 41667, 52310, 49870, 57119, 29921, 90459, 51477, 65998, 21101, 48881, 8979, 48160, 83567, 63026, 38282, 92997, 62957, 17886, 26509, 24005, 79325, 79035, 83440, 34717, 50500, 13538, 59858, 81073, 24857, 30673, 89348, 4047, 15922, 91091, 38555, 33409, 5234, 18128, 16223, 59575, 71812, 44240, 33120, 75605, 20468, 3330, 9157, 28535, 48186, 29608, 56843, 96376, 64520, 83903, 30693, 87832, 15506], dtype=np.int32)

_NW = 32
_CROWS = 1024
_CPW = _CROWS // _NW
_RPW = _BATCH // 16

@functools.cache
def _make_sc_gather():
    mesh = plsc.VectorSubcoreMesh(core_axis_name="c", subcore_axis_name="s")

    @functools.partial(
        pl.kernel,
        mesh=mesh,
        out_type=[
            jax.ShapeDtypeStruct((_CROWS, _DIM + 1), jnp.float32),
            jax.ShapeDtypeStruct((_BATCH, _DIM), jnp.float32),
        ],
        scratch_types=[
            pltpu.VMEM((_CPW,), jnp.int32),
            pltpu.VMEM((_CPW, _DIM + 1), jnp.float32),
            pltpu.VMEM((_RPW,), jnp.int32),
            pltpu.VMEM((_RPW, _DIM), jnp.float32),
            pltpu.SemaphoreType.DMA,
        ],
    )
    def _sc_gather(class_hbm, rel_hbm, cidx_hbm, ridx_hbm, out_c, out_r,
                   cidx_v, crows_v, ridx_v, rrows_v, sem):
        wid = lax.axis_index("s") * 2 + lax.axis_index("c")
        base = wid * _CPW
        pltpu.sync_copy(cidx_hbm.at[pl.ds(base, _CPW)], cidx_v)

        for c in range(_CPW // 16):
            v = cidx_v[pl.ds(c * 16, 16)]
            for j in range(16):
                idx = v[j]
                pltpu.sync_copy(class_hbm.at[pl.ds(idx, 1)],
                                crows_v.at[pl.ds(c * 16 + j, 1)])
        pltpu.sync_copy(crows_v, out_c.at[pl.ds(base, _CPW)])

        @pl.when(wid < 16)
        def _():
            rbase = wid * _RPW
            pltpu.sync_copy(ridx_hbm.at[pl.ds(rbase, _RPW)], ridx_v)
            pltpu.async_copy(rel_hbm.at[ridx_v], rrows_v, sem).wait()
            pltpu.sync_copy(rrows_v, out_r.at[pl.ds(rbase, _RPW)])

    return _sc_gather


def _norm(x):
    return jnp.sqrt(jnp.sum(x * x, axis=1, keepdims=True))


def _math_body(g_ref, rel_ref, out_ref):
    g = g_ref[...]
    cc = g[:, :_DIM]
    r = jnp.abs(g[:, _DIM:_DIM + 1])

    def grp(i):
        return cc[i * _BATCH:(i + 1) * _BATCH], r[i * _BATCH:(i + 1) * _BATCH]

    c1, rc1 = grp(0)
    d1, rd1 = grp(1)
    c2, rc2 = grp(2)
    d2, rd2 = grp(3)
    e2, re2 = grp(4)
    c3, rc3 = grp(5)
    d3, rd3 = grp(6)
    rel = rel_ref[...]

    reg1 = jnp.abs(_norm(c1) - 1.0) + jnp.abs(_norm(d1) - 1.0)
    l1 = jax.nn.relu(_norm(c1 - d1) + rc1 - rd1)
    loss1 = jnp.sum(jnp.sqrt(l1 + reg1)) / _BATCH

    reg2 = (jnp.abs(_norm(c2) - 1.0) + jnp.abs(_norm(d2) - 1.0)
            + jnp.abs(_norm(e2) - 1.0))
    l2 = (jax.nn.relu(_norm(c2 - d2) - rc2 - rd2)
          + jax.nn.relu(_norm(c2 - e2) - rc2)
          + jax.nn.relu(_norm(d2 - e2) - rd2))
    loss2 = jnp.sum(jnp.sqrt(l2 + reg2)) / _BATCH

    reg3 = jnp.abs(_norm(c3) - 1.0) + jnp.abs(_norm(d3) - 1.0)
    l3 = jnp.maximum(0.0, _norm(c3 + rel - d3) + rc3 - rd3)
    loss3 = jnp.sum(jnp.sqrt(l3 + reg3)) / _BATCH

    out_ref[...] = jnp.broadcast_to(loss1 + loss2 + loss3, (1, 1))


def kernel(nf1, nf2, nf3, class_emb, rel_emb):
    d1 = nf1[_IDX1]
    d2 = nf2[_IDX2]
    d3 = nf3[_IDX3]
    cidx = jnp.concatenate([
        d1[:, 0], d1[:, 1],
        d2[:, 0], d2[:, 1], d2[:, 2],
        d3[:, 0], d3[:, 2],
        jnp.zeros((_CROWS - 7 * _BATCH,), jnp.int32),
    ])
    ridx = d3[:, 1]
    gc, gr = _make_sc_gather()(class_emb, rel_emb, cidx, ridx)
    out = pl.pallas_call(
        _math_body,
        out_shape=jax.ShapeDtypeStruct((1, 1), jnp.float32),
    )(gc, gr)
    return jnp.reshape(out, ())
